# baseline (speedup 1.0000x reference)
import functools

import jax
import jax.numpy as jnp
import numpy as np
from jax import lax
from jax.experimental import pallas as pl
from jax.experimental.pallas import tpu as pltpu
from jax.experimental.pallas import tpu_sc as plsc

P = 96
V = 192
C = 2
K = 27
PLANE = V * V
SLAB = P * P
NW = 32
PPW = V // NW


def _gauss_axis() -> np.ndarray:
    c = (P - 1) / 2.0
    sigma = 0.125 * P
    x = np.arange(P, dtype=np.float64)
    g = np.exp(-0.5 * ((x - c) / sigma) ** 2)
    return g / g.max()


def _upsample_matrix() -> np.ndarray:
    U = np.zeros((P, V), dtype=np.float64)
    for x in range(V):
        s = x / 2.0 - 0.25
        lo = int(np.floor(s))
        f = s - lo
        for idx, w in ((lo, 1.0 - f), (lo + 1, f)):
            if 0 <= idx < P:
                U[idx, x] += w
        U[:, x] /= U[:, x].sum()
    return U


_GN = _gauss_axis()
_GV = np.asarray(_GN, dtype=np.float32)
_HW_WT = np.asarray(np.outer(_GN, _GN).reshape(SLAB), dtype=np.float32)
_U = np.asarray(_upsample_matrix(), dtype=np.float32)


def _sc_body(prows, meta, gvec, acc_out,
             acc0_v, acc1_v, slab_v, meta_v, gv_v, wl, sem0, sem1):
    wid = lax.axis_index("s") * 2 + lax.axis_index("c")
    pltpu.sync_copy(meta, meta_v)
    pltpu.sync_copy(gvec, gv_v.at[pl.ds(0, P)])
    zeros16 = jnp.zeros((16,), jnp.float32)
    gx = [gv_v[pl.ds(16 * xv, 16)] for xv in range(P // 16)]

    def issue(j, buf, sem):
        i = wl[4 * j]
        dz = wl[4 * j + 1]
        pltpu.async_copy(prows.at[i, 0, dz], slab_v.at[buf, 0], sem)
        pltpu.async_copy(prows.at[i, 1, dz], slab_v.at[buf, 1], sem)

    def wait_pair(buf, sem):
        pltpu.make_async_copy(prows.at[0, 0, 0], slab_v.at[buf, 0], sem).wait()
        pltpu.make_async_copy(prows.at[0, 0, 0], slab_v.at[buf, 1], sem).wait()

    def accum(j, buf):
        dz = wl[4 * j + 1]
        hi = wl[4 * j + 2]
        wi = wl[4 * j + 3]
        gd = gv_v[pl.ds(dz, 16)][0]

        @plsc.parallel_loop(0, P, unroll=8)
        def _(y):
            s = gd * gv_v[pl.ds(y, 16)][0]
            ba = (hi + y) * V + wi
            for xv in range(P // 16):
                o = 16 * xv
                w16 = gx[xv] * s
                plsc.addupdate(acc0_v.at[pl.ds(ba + o, 16)],
                               slab_v[buf, 0, y, pl.ds(o, 16)] * w16)
                plsc.addupdate(acc1_v.at[pl.ds(ba + o, 16)],
                               slab_v[buf, 1, y, pl.ds(o, 16)] * w16)

    def plane_body(t, _):
        d = wid + NW * t

        def build(i, n):
            mrow = meta_v[i, :]
            di = mrow[0]
            cover = jnp.logical_and(di <= d, d < di + P)

            @pl.when(cover)
            def _():
                wl[4 * n] = i
                wl[4 * n + 1] = d - di
                wl[4 * n + 2] = mrow[1]
                wl[4 * n + 3] = mrow[2]
            return n + jnp.where(cover, 1, 0)
        n = lax.fori_loop(0, K, build, 0)

        @plsc.parallel_loop(0, PLANE // 16, unroll=8)
        def _(j):
            acc0_v[pl.ds(j * 16, 16)] = zeros16
            acc1_v[pl.ds(j * 16, 16)] = zeros16

        @pl.when(n > 0)
        def _():
            issue(0, 0, sem0)

        def step(j, _):
            @pl.when(j % 2 == 0)
            def _():
                @pl.when(j + 1 < n)
                def _():
                    issue(j + 1, 1, sem1)
                wait_pair(0, sem0)
                accum(j, 0)

            @pl.when(j % 2 == 1)
            def _():
                @pl.when(j + 1 < n)
                def _():
                    issue(j + 1, 0, sem0)
                wait_pair(1, sem1)
                accum(j, 1)
            return 0
        lax.fori_loop(0, n, step, 0)
        pltpu.sync_copy(acc0_v, acc_out.at[d])
        pltpu.sync_copy(acc1_v, acc_out.at[V + d])
        return 0
    lax.fori_loop(0, PPW, plane_body, 0)


def _sc_aggregate(prows, meta):
    mesh = plsc.VectorSubcoreMesh(core_axis_name="c", subcore_axis_name="s")
    f = pl.kernel(
        _sc_body,
        out_type=jax.ShapeDtypeStruct((C * V, PLANE), jnp.float32),
        mesh=mesh,
        scratch_types=[
            pltpu.VMEM((PLANE,), jnp.float32),
            pltpu.VMEM((PLANE,), jnp.float32),
            pltpu.VMEM((2, C, P, P), jnp.float32),
            pltpu.VMEM((K, 16), jnp.int32),
            pltpu.VMEM((P + 16,), jnp.float32),
            pltpu.SMEM((128,), jnp.int32),
            pltpu.SemaphoreType.DMA,
            pltpu.SemaphoreType.DMA,
        ],
    )
    return f(prows, meta, jnp.asarray(_GV))


def _tc_finalize_body(acc_ref, gdt_ref, glob_ref, u_ref, gh_ref, gw_ref,
                      out_ref):
    d = pl.program_id(0)
    k = d // 2
    odd = d % 2
    za = jnp.clip(jnp.where(odd == 0, k - 1, k), 0, P - 1)
    zb = jnp.clip(jnp.where(odd == 0, k, k + 1), 0, P - 1)
    wa = jnp.where(odd == 0, jnp.float32(0.25), jnp.float32(0.75))
    pa = glob_ref[:, pl.ds(za, 1), :, :].reshape(C, P, P)
    pb = glob_ref[:, pl.ds(zb, 1), :, :].reshape(C, P, P)
    pd_ = wa * pa + (1.0 - wa) * pb
    u = u_ref[...]
    tmp = lax.dot_general(pd_, u, (((2,), (0,)), ((), ())),
                          preferred_element_type=jnp.float32)
    ug = lax.dot_general(tmp, u, (((1,), (0,)), ((), ())),
                         preferred_element_type=jnp.float32)
    ug = jnp.swapaxes(ug, 1, 2)
    gdrow = gdt_ref[pl.ds(d, 1), :]
    a = gh_ref[...] * gdrow
    wt = lax.dot_general(a, gw_ref[...], (((1,), (0,)), ((), ())),
                         preferred_element_type=jnp.float32)
    wt = wt.reshape(1, V, V)
    acc = acc_ref[...].reshape(C, V, V)
    res = acc / (wt + 1e-20) + jnp.where(wt > 0.0, 0.0, ug)
    out_ref[...] = res.reshape(C, 1, V, V)


def _tc_finalize(acc, glob, gdt, gh, gw):
    return pl.pallas_call(
        _tc_finalize_body,
        grid=(V,),
        in_specs=[
            pl.BlockSpec((C, 1, V, V), lambda d: (0, d, 0, 0)),
            pl.BlockSpec((V, K), lambda d: (0, 0)),
            pl.BlockSpec((C, P, P, P), lambda d: (0, 0, 0, 0)),
            pl.BlockSpec((P, V), lambda d: (0, 0)),
            pl.BlockSpec((V, K), lambda d: (0, 0)),
            pl.BlockSpec((K, V), lambda d: (0, 0)),
        ],
        out_specs=pl.BlockSpec((C, 1, V, V), lambda d: (0, d, 0, 0)),
        out_shape=jax.ShapeDtypeStruct((C, V, V, V), jnp.float32),
    )(acc, gdt, glob, jnp.asarray(_U), gh, gw)


def kernel(patch_logits, global_logit, slice_meta):
    prows = patch_logits
    meta = jnp.zeros((K, 16), jnp.int32).at[:, :3].set(slice_meta)
    acc = _sc_aggregate(prows, meta)
    ga = jnp.asarray(_GV)
    j = jnp.arange(V)[None, :]
    st = slice_meta.astype(jnp.int32)
    def shifted(s):
        s = s[:, None]
        inb = jnp.logical_and(j >= s, j < s + P)
        return jnp.where(inb, ga[jnp.clip(j - s, 0, P - 1)], 0.0)
    gdm = shifted(st[:, 0])
    gh = shifted(st[:, 1])
    gw = shifted(st[:, 2])
    out = _tc_finalize(acc.reshape(C, V, V, V), global_logit[0],
                       gdm.T, gh.T, gw)
    return out[None]


# speedup vs baseline: 1.0425x; 1.0425x over previous
import functools

import jax
import jax.numpy as jnp
import numpy as np
from jax import lax
from jax.experimental import pallas as pl
from jax.experimental.pallas import tpu as pltpu
from jax.experimental.pallas import tpu_sc as plsc

P = 96
V = 192
C = 2
K = 27
PLANE = V * V
SLAB = P * P
NW = 32
PPW = V // NW


def _gauss_axis() -> np.ndarray:
    c = (P - 1) / 2.0
    sigma = 0.125 * P
    x = np.arange(P, dtype=np.float64)
    g = np.exp(-0.5 * ((x - c) / sigma) ** 2)
    return g / g.max()


def _upsample_matrix() -> np.ndarray:
    U = np.zeros((P, V), dtype=np.float64)
    for x in range(V):
        s = x / 2.0 - 0.25
        lo = int(np.floor(s))
        f = s - lo
        for idx, w in ((lo, 1.0 - f), (lo + 1, f)):
            if 0 <= idx < P:
                U[idx, x] += w
        U[:, x] /= U[:, x].sum()
    return U


_GN = _gauss_axis()
_GV = np.asarray(_GN, dtype=np.float32)
_HW_WT = np.asarray(np.outer(_GN, _GN).reshape(SLAB), dtype=np.float32)
_U = np.asarray(_upsample_matrix(), dtype=np.float32)


def _sc_body(prows, meta, gvec, acc_out,
             acc0_v, acc1_v, slab_v, meta_v, gv_v, wl, sem0, sem1):
    wid = lax.axis_index("s") * 2 + lax.axis_index("c")
    pltpu.sync_copy(meta, meta_v)
    pltpu.sync_copy(gvec, gv_v.at[pl.ds(0, P)])
    zeros16 = jnp.zeros((16,), jnp.float32)
    gx = [gv_v[pl.ds(16 * xv, 16)] for xv in range(P // 16)]

    def issue(j, buf, sem):
        i = wl[4 * j]
        dz = wl[4 * j + 1]
        pltpu.async_copy(prows.at[i, 0, dz], slab_v.at[buf, 0], sem)
        pltpu.async_copy(prows.at[i, 1, dz], slab_v.at[buf, 1], sem)

    def wait_pair(buf, sem):
        pltpu.make_async_copy(prows.at[0, 0, 0], slab_v.at[buf, 0], sem).wait()
        pltpu.make_async_copy(prows.at[0, 0, 0], slab_v.at[buf, 1], sem).wait()

    def accum(j, buf):
        dz = wl[4 * j + 1]
        hi = wl[4 * j + 2]
        wi = wl[4 * j + 3]
        gd = gv_v[pl.ds(dz, 16)][0]

        @plsc.parallel_loop(0, P, unroll=4)
        def _(y):
            s = gd * gv_v[pl.ds(y, 16)][0]
            ba = (hi + y) * V + wi
            for xv in range(P // 16):
                o = 16 * xv
                w16 = gx[xv] * s
                plsc.addupdate(acc0_v.at[pl.ds(ba + o, 16)],
                               slab_v[buf, 0, y, pl.ds(o, 16)] * w16)
                plsc.addupdate(acc1_v.at[pl.ds(ba + o, 16)],
                               slab_v[buf, 1, y, pl.ds(o, 16)] * w16)

    def plane_body(t, _):
        d = wid + NW * t

        def build(i, n):
            mrow = meta_v[i, :]
            di = mrow[0]
            cover = jnp.logical_and(di <= d, d < di + P)

            @pl.when(cover)
            def _():
                wl[4 * n] = i
                wl[4 * n + 1] = d - di
                wl[4 * n + 2] = mrow[1]
                wl[4 * n + 3] = mrow[2]
            return n + jnp.where(cover, 1, 0)
        n = lax.fori_loop(0, K, build, 0)

        @plsc.parallel_loop(0, PLANE // 16, unroll=8)
        def _(j):
            acc0_v[pl.ds(j * 16, 16)] = zeros16
            acc1_v[pl.ds(j * 16, 16)] = zeros16

        @pl.when(n > 0)
        def _():
            issue(0, 0, sem0)

        def step(j, _):
            @pl.when(j % 2 == 0)
            def _():
                @pl.when(j + 1 < n)
                def _():
                    issue(j + 1, 1, sem1)
                wait_pair(0, sem0)
                accum(j, 0)

            @pl.when(j % 2 == 1)
            def _():
                @pl.when(j + 1 < n)
                def _():
                    issue(j + 1, 0, sem0)
                wait_pair(1, sem1)
                accum(j, 1)
            return 0
        lax.fori_loop(0, n, step, 0)
        pltpu.sync_copy(acc0_v, acc_out.at[d])
        pltpu.sync_copy(acc1_v, acc_out.at[V + d])
        return 0
    lax.fori_loop(0, PPW, plane_body, 0)


def _sc_aggregate(prows, meta):
    mesh = plsc.VectorSubcoreMesh(core_axis_name="c", subcore_axis_name="s")
    f = pl.kernel(
        _sc_body,
        out_type=jax.ShapeDtypeStruct((C * V, PLANE), jnp.float32),
        mesh=mesh,
        scratch_types=[
            pltpu.VMEM((PLANE,), jnp.float32),
            pltpu.VMEM((PLANE,), jnp.float32),
            pltpu.VMEM((2, C, P, P), jnp.float32),
            pltpu.VMEM((K, 16), jnp.int32),
            pltpu.VMEM((P + 16,), jnp.float32),
            pltpu.SMEM((128,), jnp.int32),
            pltpu.SemaphoreType.DMA,
            pltpu.SemaphoreType.DMA,
        ],
    )
    return f(prows, meta, jnp.asarray(_GV))


def _tc_finalize_body(acc_ref, gdt_ref, glob_ref, u_ref, gh_ref, gw_ref,
                      out_ref):
    d = pl.program_id(0)
    k = d // 2
    odd = d % 2
    za = jnp.clip(jnp.where(odd == 0, k - 1, k), 0, P - 1)
    zb = jnp.clip(jnp.where(odd == 0, k, k + 1), 0, P - 1)
    wa = jnp.where(odd == 0, jnp.float32(0.25), jnp.float32(0.75))
    pa = glob_ref[:, pl.ds(za, 1), :, :].reshape(C, P, P)
    pb = glob_ref[:, pl.ds(zb, 1), :, :].reshape(C, P, P)
    pd_ = wa * pa + (1.0 - wa) * pb
    u = u_ref[...]
    tmp = lax.dot_general(pd_, u, (((2,), (0,)), ((), ())),
                          preferred_element_type=jnp.float32)
    ug = lax.dot_general(tmp, u, (((1,), (0,)), ((), ())),
                         preferred_element_type=jnp.float32)
    ug = jnp.swapaxes(ug, 1, 2)
    gdrow = gdt_ref[pl.ds(d, 1), :]
    a = gh_ref[...] * gdrow
    wt = lax.dot_general(a, gw_ref[...], (((1,), (0,)), ((), ())),
                         preferred_element_type=jnp.float32)
    wt = wt.reshape(1, V, V)
    acc = acc_ref[...].reshape(C, V, V)
    res = acc / (wt + 1e-20) + jnp.where(wt > 0.0, 0.0, ug)
    out_ref[...] = res.reshape(C, 1, V, V)


def _tc_finalize(acc, glob, gdt, gh, gw):
    return pl.pallas_call(
        _tc_finalize_body,
        grid=(V,),
        in_specs=[
            pl.BlockSpec((C, 1, V, V), lambda d: (0, d, 0, 0)),
            pl.BlockSpec((V, K), lambda d: (0, 0)),
            pl.BlockSpec((C, P, P, P), lambda d: (0, 0, 0, 0)),
            pl.BlockSpec((P, V), lambda d: (0, 0)),
            pl.BlockSpec((V, K), lambda d: (0, 0)),
            pl.BlockSpec((K, V), lambda d: (0, 0)),
        ],
        out_specs=pl.BlockSpec((C, 1, V, V), lambda d: (0, d, 0, 0)),
        out_shape=jax.ShapeDtypeStruct((C, V, V, V), jnp.float32),
    )(acc, gdt, glob, jnp.asarray(_U), gh, gw)


def kernel(patch_logits, global_logit, slice_meta):
    prows = patch_logits
    meta = jnp.zeros((K, 16), jnp.int32).at[:, :3].set(slice_meta)
    acc = _sc_aggregate(prows, meta)
    ga = jnp.asarray(_GV)
    j = jnp.arange(V)[None, :]
    st = slice_meta.astype(jnp.int32)
    def shifted(s):
        s = s[:, None]
        inb = jnp.logical_and(j >= s, j < s + P)
        return jnp.where(inb, ga[jnp.clip(j - s, 0, P - 1)], 0.0)
    gdm = shifted(st[:, 0])
    gh = shifted(st[:, 1])
    gw = shifted(st[:, 2])
    out = _tc_finalize(acc.reshape(C, V, V, V), global_logit[0],
                       gdm.T, gh.T, gw)
    return out[None]
